# Initial kernel scaffold; baseline (speedup 1.0000x reference)
import jax
import jax.numpy as jnp
from jax.experimental import pallas as pl


def kernel(out, z_rv, x_rv_batch):
    raise NotImplementedError("write your pallas kernel here")


# speedup vs baseline: 181.4690x; 181.4690x over previous
import functools

import jax
import jax.numpy as jnp
from jax import lax
from jax.experimental import pallas as pl
from jax.experimental.pallas import tpu as pltpu
from jax.experimental.pallas import tpu_sc as plsc

N = 10000
D = 128
NUM_Z = 100
NUM_GRAPHS = 64

NC = 2
NS = 16
LANES = 16

PER_W = 640
CHUNK = 64
NCHUNK = PER_W // CHUNK
HALF = (NUM_GRAPHS // NC) * NUM_Z
ACC_ROWS = 3328
TRASH = HALF
ZROWS = 16
ZSLABS = (ACC_ROWS // NS) // ZROWS
OUT_PER_S = HALF // NS


def _pool_body(out_hbm, z_hbm, b_hbm, pooled_hbm,
               rows_v, z_v, b_v, idx_v, zero_v, acc_sh):
    c = lax.axis_index("c")
    s = lax.axis_index("s")

    base = jnp.minimum(s * PER_W, N - PER_W).astype(jnp.int32)

    pltpu.sync_copy(out_hbm.at[pl.ds(base, PER_W)], rows_v)
    pltpu.sync_copy(z_hbm.at[pl.ds(base, PER_W)], z_v)
    pltpu.sync_copy(b_hbm.at[pl.ds(base, PER_W)], b_v)

    zvec = jnp.zeros((LANES,), jnp.float32)
    for r in range(ZROWS):
        for k in range(D // LANES):
            zero_v[r, pl.ds(k * LANES, LANES)] = zvec
    for t in range(ZSLABS):
        pltpu.sync_copy(zero_v, acc_sh.at[pl.ds(s * (ZSLABS * ZROWS) + t * ZROWS, ZROWS)])

    own_lo = s * PER_W
    half_lo = c * HALF
    for j in range(PER_W // LANES):
        off = j * LANES
        gid = base + off + lax.iota(jnp.int32, LANES)
        zv = z_v[pl.ds(off, LANES)]
        bv = b_v[pl.ds(off, LANES)]
        local = bv * NUM_Z + zv - 1 - half_lo
        valid = (gid >= own_lo) & (local >= 0) & (local < HALF)
        idx_v[j // (CHUNK // LANES),
              pl.ds((j % (CHUNK // LANES)) * LANES, LANES)] = jnp.where(valid, local, TRASH)

    plsc.subcore_barrier()

    for t in range(NCHUNK):
        pltpu.sync_copy(rows_v.at[pl.ds(t * CHUNK, CHUNK)],
                        acc_sh.at[idx_v.at[t]], add=True)

    plsc.subcore_barrier()

    pltpu.sync_copy(acc_sh.at[pl.ds(s * OUT_PER_S, OUT_PER_S)],
                    pooled_hbm.at[pl.ds(c * HALF + s * OUT_PER_S, OUT_PER_S)])


@jax.jit
def _pooled(out, z_rv, x_rv_batch):
    mesh = plsc.VectorSubcoreMesh(core_axis_name="c", subcore_axis_name="s")
    run = pl.kernel(
        _pool_body,
        out_type=jax.ShapeDtypeStruct((NUM_GRAPHS * NUM_Z, D), jnp.float32),
        mesh=mesh,
        scratch_types=[
            pltpu.VMEM((PER_W, D), jnp.float32),
            pltpu.VMEM((PER_W,), jnp.int32),
            pltpu.VMEM((PER_W,), jnp.int32),
            pltpu.VMEM((NCHUNK, CHUNK), jnp.int32),
            pltpu.VMEM((ZROWS, D), jnp.float32),
            pltpu.VMEM_SHARED((ACC_ROWS, D), jnp.float32),
        ],
    )
    return run(out, z_rv, x_rv_batch)


def kernel(out, z_rv, x_rv_batch):
    pooled = _pooled(out, z_rv, x_rv_batch)
    return pooled.reshape(NUM_GRAPHS, NUM_Z * D)


# speedup vs baseline: 204.2461x; 1.1255x over previous
import jax
import jax.numpy as jnp
from jax import lax
from jax.experimental import pallas as pl
from jax.experimental.pallas import tpu as pltpu
from jax.experimental.pallas import tpu_sc as plsc

N = 10000
D = 128
NUM_Z = 100
NUM_GRAPHS = 64

NC = 2
NS = 16
LANES = 16

PER_W = 640
CHUNK = 128
NCHUNK = PER_W // CHUNK
HALF = (NUM_GRAPHS // NC) * NUM_Z
ACC_ROWS = 3328
TRASH = HALF
ZROWS = 16
ZSLABS = (ACC_ROWS // NS) // ZROWS
OUT_PER_S = HALF // NS


def _pool_body(out_hbm, z_hbm, b_hbm, pooled_hbm,
               rows_v, z_v, b_v, idx_v, zero_v, acc_sh, ld_sem, sc_sem):
    c = lax.axis_index("c")
    s = lax.axis_index("s")

    base = jnp.minimum(s * PER_W, N - PER_W).astype(jnp.int32)

    pltpu.sync_copy(z_hbm.at[pl.ds(base, PER_W)], z_v)
    pltpu.sync_copy(b_hbm.at[pl.ds(base, PER_W)], b_v)

    own_lo = s * PER_W
    half_lo = c * HALF
    for t in range(NCHUNK):
        trash = TRASH + s * NCHUNK + t
        for jj in range(CHUNK // LANES):
            off = t * CHUNK + jj * LANES
            gid = base + off + lax.iota(jnp.int32, LANES)
            zv = z_v[pl.ds(off, LANES)]
            bv = b_v[pl.ds(off, LANES)]
            local = bv * NUM_Z + zv - 1 - half_lo
            valid = (gid >= own_lo) & (local >= 0) & (local < HALF)
            idx_v[t, pl.ds(jj * LANES, LANES)] = jnp.where(valid, local, trash)

    loads = [pltpu.make_async_copy(out_hbm.at[pl.ds(base + t * CHUNK, CHUNK)],
                                   rows_v.at[pl.ds(t * CHUNK, CHUNK)], ld_sem)
             for t in range(NCHUNK)]
    for t in range(NCHUNK):
        loads[t].start()

    zvec = jnp.zeros((LANES,), jnp.float32)
    for r in range(ZROWS):
        for k in range(D // LANES):
            zero_v[r, pl.ds(k * LANES, LANES)] = zvec
    for t in range(ZSLABS):
        pltpu.sync_copy(zero_v, acc_sh.at[pl.ds(s * (ZSLABS * ZROWS) + t * ZROWS, ZROWS)])

    for t in range(NCHUNK):
        loads[t].wait()

    plsc.subcore_barrier()

    scats = [pltpu.make_async_copy(rows_v.at[pl.ds(t * CHUNK, CHUNK)],
                                   acc_sh.at[idx_v.at[t]], sc_sem)
             for t in range(NCHUNK)]
    for t in range(NCHUNK):
        scats[t].start(add=True)
    for t in range(NCHUNK):
        scats[t].wait()

    plsc.subcore_barrier()

    pltpu.sync_copy(acc_sh.at[pl.ds(s * OUT_PER_S, OUT_PER_S)],
                    pooled_hbm.at[pl.ds(c * HALF + s * OUT_PER_S, OUT_PER_S)])


@jax.jit
def _pooled(out, z_rv, x_rv_batch):
    mesh = plsc.VectorSubcoreMesh(core_axis_name="c", subcore_axis_name="s")
    run = pl.kernel(
        _pool_body,
        out_type=jax.ShapeDtypeStruct((NUM_GRAPHS * NUM_Z, D), jnp.float32),
        mesh=mesh,
        scratch_types=[
            pltpu.VMEM((PER_W, D), jnp.float32),
            pltpu.VMEM((PER_W,), jnp.int32),
            pltpu.VMEM((PER_W,), jnp.int32),
            pltpu.VMEM((NCHUNK, CHUNK), jnp.int32),
            pltpu.VMEM((ZROWS, D), jnp.float32),
            pltpu.VMEM_SHARED((ACC_ROWS, D), jnp.float32),
            pltpu.SemaphoreType.DMA,
            pltpu.SemaphoreType.DMA,
        ],
    )
    return run(out, z_rv, x_rv_batch)


def kernel(out, z_rv, x_rv_batch):
    pooled = _pooled(out, z_rv, x_rv_batch)
    return pooled.reshape(NUM_GRAPHS, NUM_Z * D)
